# baseline (speedup 1.0000x reference)
import functools

import jax
import jax.numpy as jnp
from jax import lax
from jax.experimental import pallas as pl
from jax.experimental.pallas import tpu as pltpu
from jax.experimental.pallas import tpu_sc as plsc

N = 10000
E = 320000
D = 128
H = 16
C = 32
K = 10

NC = 2
NS = 16
HALF = C // NC

NPAD = 10240
SLICE = NPAD // NS
TRASH = N

CHUNK = 128
CHUNKS = 160
EPT = CHUNK * CHUNKS
PADE = NS * EPT

_mesh = plsc.VectorSubcoreMesh(core_axis_name="c", subcore_axis_name="s")
_sc_params = pltpu.CompilerParams(use_tc_tiling_on_sc=False)


def _deg_body(src_hbm, dst_hbm, deg_hbm, idx_v, ones_v, zbuf, obuf, acc_sh,
              dsem):
    c = lax.axis_index("c")
    s = lax.axis_index("s")
    base = s * SLICE

    @pl.loop(0, SLICE)
    def _(i):
        zbuf[i] = jnp.zeros((HALF,), jnp.float32)

    @pl.loop(0, CHUNK)
    def _(i):
        ones_v[i] = jnp.ones((HALF,), jnp.float32)

    @pl.when(c == 0)
    def _():
        pltpu.sync_copy(src_hbm.at[s], idx_v)

    @pl.when(c == 1)
    def _():
        pltpu.sync_copy(dst_hbm.at[s], idx_v)

    pltpu.sync_copy(zbuf, acc_sh.at[pl.ds(base, SLICE)])
    plsc.subcore_barrier()

    @pl.loop(0, CHUNKS)
    def _(j):
        pltpu.async_copy(ones_v, acc_sh.at[idx_v.at[j]], dsem, add=True)

    @pl.loop(0, CHUNKS)
    def _(j):
        pltpu.make_async_copy(ones_v, acc_sh.at[idx_v.at[j]], dsem).wait()

    plsc.subcore_barrier()
    pltpu.sync_copy(acc_sh.at[pl.ds(base, SLICE)], obuf)
    pltpu.sync_copy(obuf, deg_hbm.at[c, pl.ds(base, SLICE)])


@jax.jit
def _deg_call(src_t, dst_t):
    return pl.kernel(
        _deg_body,
        out_type=jax.ShapeDtypeStruct((NC, NPAD, HALF), jnp.float32),
        mesh=_mesh,
        compiler_params=_sc_params,
        scratch_types=[
            pltpu.VMEM((CHUNKS, CHUNK), jnp.int32),
            pltpu.VMEM((CHUNK, HALF), jnp.float32),
            pltpu.VMEM((SLICE, HALF), jnp.float32),
            pltpu.VMEM((SLICE, HALF), jnp.float32),
            pltpu.VMEM_SHARED((NPAD, HALF), jnp.float32),
            pltpu.SemaphoreType.DMA,
        ],
    )(src_t, dst_t)


def _prep_body(x_ref, w0_ref, w1_ref, deg_ref, hl_ref, s0_ref, dd_ref, db_ref):
    z1 = jax.nn.relu(jnp.dot(x_ref[...], w0_ref[...],
                             preferred_element_type=jnp.float32))
    hl = jnp.dot(z1, w1_ref[...], preferred_element_type=jnp.float32)
    dsrc = lax.rsqrt(jnp.clip(deg_ref[0][:, 0:1], 1.0))
    ddst = lax.rsqrt(jnp.clip(deg_ref[1][:, 0:1], 1.0))
    hl_ref[...] = hl
    s0_ref[...] = hl * dsrc
    ones = jnp.ones((1, HALF), jnp.float32)
    dd_ref[...] = ddst * ones
    db_ref[...] = (dsrc * ddst) * ones


@jax.jit
def _prep_call(xp, W0, W1, degs):
    nb = 2048
    grid = NPAD // nb
    return pl.pallas_call(
        _prep_body,
        grid=(grid,),
        in_specs=[
            pl.BlockSpec((nb, D), lambda i: (i, 0)),
            pl.BlockSpec((D, H), lambda i: (0, 0)),
            pl.BlockSpec((H, C), lambda i: (0, 0)),
            pl.BlockSpec((NC, nb, HALF), lambda i: (0, i, 0)),
        ],
        out_specs=[
            pl.BlockSpec((nb, C), lambda i: (i, 0)),
            pl.BlockSpec((nb, C), lambda i: (i, 0)),
            pl.BlockSpec((nb, HALF), lambda i: (i, 0)),
            pl.BlockSpec((nb, HALF), lambda i: (i, 0)),
        ],
        out_shape=[
            jax.ShapeDtypeStruct((NPAD, C), jnp.float32),
            jax.ShapeDtypeStruct((NPAD, C), jnp.float32),
            jax.ShapeDtypeStruct((NPAD, HALF), jnp.float32),
            jax.ShapeDtypeStruct((NPAD, HALF), jnp.float32),
        ],
    )(xp, W0, W1, degs)


GCHUNK = 256
GCHUNKS = EPT // GCHUNK
NBUF = 8
NBLK = GCHUNKS // NBUF
assert NBUF * NBLK == GCHUNKS


def _hops_body(srcg_hbm, dst_hbm, s0_hbm, dd_hbm, db_hbm, p_hbm, tabh_hbm,
               sidx, didx, rows, accb, pbuf, dd, db, zbuf, acc_sh,
               gsem, ssem):
    c = lax.axis_index("c")
    s = lax.axis_index("s")
    base = s * SLICE

    pltpu.sync_copy(srcg_hbm.at[s], sidx)
    pltpu.sync_copy(dst_hbm.at[s], didx)
    pltpu.sync_copy(dd_hbm.at[pl.ds(base, SLICE)], dd)
    pltpu.sync_copy(db_hbm.at[pl.ds(base, SLICE)], db)

    @pl.loop(0, SLICE // 2)
    def _(i):
        zbuf[i] = jnp.zeros((HALF,), jnp.float32)

    pltpu.sync_copy(s0_hbm.at[pl.ds(base, SLICE), pl.ds(c * HALF, HALF)], accb)
    pltpu.sync_copy(accb, tabh_hbm.at[0, c, pl.ds(base, SLICE)])

    def zero_acc():
        pltpu.sync_copy(zbuf, acc_sh.at[pl.ds(base, SLICE // 2)])
        pltpu.sync_copy(zbuf, acc_sh.at[pl.ds(base + SLICE // 2, SLICE // 2)])

    zero_acc()
    plsc.subcore_barrier()

    for k in range(1, K + 1):
        rp, wp = (k - 1) % 2, k % 2
        tab = tabh_hbm.at[rp, c]

        for b in range(NBUF):
            pltpu.async_copy(tab.at[sidx.at[b]], rows.at[b], gsem.at[b])

        @pl.loop(0, NBLK)
        def _(blk):
            j0 = blk * NBUF
            for b in range(NBUF):
                pltpu.make_async_copy(
                    tab.at[sidx.at[j0 + b]], rows.at[b], gsem.at[b]).wait()
                for h in range(2):
                    pltpu.async_copy(
                        rows.at[b].at[pl.ds(h * CHUNK, CHUNK)],
                        acc_sh.at[didx.at[2 * (j0 + b) + h]], ssem.at[b],
                        add=True)
            for b in range(NBUF):
                for h in range(2):
                    pltpu.make_async_copy(
                        rows.at[b].at[pl.ds(h * CHUNK, CHUNK)],
                        acc_sh.at[didx.at[2 * (j0 + b) + h]],
                        ssem.at[b]).wait()

                @pl.when(blk < NBLK - 1)
                def _():
                    pltpu.async_copy(
                        tab.at[sidx.at[j0 + NBUF + b]], rows.at[b],
                        gsem.at[b])

        plsc.subcore_barrier()

        pltpu.sync_copy(acc_sh.at[pl.ds(base, SLICE)], accb)

        @pl.loop(0, SLICE)
        def _(i):
            a = accb[i]
            pbuf[i] = a * dd[i]
            accb[i] = a * db[i]

        pltpu.sync_copy(
            pbuf, p_hbm.at[k - 1, pl.ds(base, SLICE), pl.ds(c * HALF, HALF)])
        pltpu.sync_copy(accb, tabh_hbm.at[wp, c, pl.ds(base, SLICE)])
        zero_acc()
        plsc.subcore_barrier()


@jax.jit
def _hops_call(src_g, dst_t, s0, dd16, db16):
    return pl.kernel(
        _hops_body,
        out_type=[
            jax.ShapeDtypeStruct((K, NPAD, C), jnp.float32),
            jax.ShapeDtypeStruct((2, NC, NPAD, HALF), jnp.float32),
        ],
        mesh=_mesh,
        compiler_params=_sc_params,
        scratch_types=[
            pltpu.VMEM((GCHUNKS, GCHUNK), jnp.int32),
            pltpu.VMEM((CHUNKS, CHUNK), jnp.int32),
            pltpu.VMEM((NBUF, GCHUNK, HALF), jnp.float32),
            pltpu.VMEM((SLICE, HALF), jnp.float32),
            pltpu.VMEM((SLICE, HALF), jnp.float32),
            pltpu.VMEM((SLICE, HALF), jnp.float32),
            pltpu.VMEM((SLICE, HALF), jnp.float32),
            pltpu.VMEM((SLICE // 2, HALF), jnp.float32),
            pltpu.VMEM_SHARED((NPAD, HALF), jnp.float32),
            pltpu.SemaphoreType.DMA((NBUF,)),
            pltpu.SemaphoreType.DMA((NBUF,)),
        ],
    )(src_g, dst_t, s0, dd16, db16)


def _attn_body(hl_ref, p_ref, wa_ref, out_ref):
    wa = wa_ref[...]
    hl = hl_ref[...]
    logits = [jnp.sum(hl * wa, axis=1, keepdims=True)]
    for k in range(K):
        logits.append(jnp.sum(p_ref[k] * wa, axis=1, keepdims=True))
    a = jnp.concatenate(logits, axis=1)
    m = jnp.max(a, axis=1, keepdims=True)
    e = jnp.exp(a - m)
    coef = e / jnp.sum(e, axis=1, keepdims=True)
    out = hl * coef[:, 0:1]
    for k in range(K):
        out = out + p_ref[k] * coef[:, k + 1:k + 2]
    out_ref[...] = out


@jax.jit
def _attn_call(hl, P, wap):
    nb = 2048
    grid = NPAD // nb
    return pl.pallas_call(
        _attn_body,
        grid=(grid,),
        in_specs=[
            pl.BlockSpec((nb, C), lambda i: (i, 0)),
            pl.BlockSpec((K, nb, C), lambda i: (0, i, 0)),
            pl.BlockSpec((nb, C), lambda i: (i, 0)),
        ],
        out_specs=pl.BlockSpec((nb, C), lambda i: (i, 0)),
        out_shape=jax.ShapeDtypeStruct((NPAD, C), jnp.float32),
    )(hl, P, wap)


def kernel(x, edge_index, W0, W1, W_attn):
    src = edge_index[0].astype(jnp.int32)
    dst = edge_index[1].astype(jnp.int32)
    pad = PADE - E
    srcp = jnp.concatenate([src, jnp.full((pad,), TRASH, jnp.int32)])
    dstp = jnp.concatenate([dst, jnp.full((pad,), TRASH, jnp.int32)])
    src_t = srcp.reshape(NS, CHUNKS, CHUNK)
    src_g = srcp.reshape(NS, GCHUNKS, GCHUNK)
    dst_t = dstp.reshape(NS, CHUNKS, CHUNK)

    xp = jnp.pad(x, ((0, NPAD - N), (0, 0)))
    wap = jnp.pad(W_attn, ((0, NPAD - N), (0, 0)))

    degs = _deg_call(src_t, dst_t)
    hl, s0, dd16, db16 = _prep_call(xp, W0, W1, degs)
    P, _ = _hops_call(src_g, dst_t, s0, dd16, db16)
    logits = _attn_call(hl, P, wap)
    return logits[:N]


# speedup vs baseline: 1.7075x; 1.7075x over previous
import functools

import jax
import jax.numpy as jnp
from jax import lax
from jax.experimental import pallas as pl
from jax.experimental.pallas import tpu as pltpu
from jax.experimental.pallas import tpu_sc as plsc

N = 10000
E = 320000
D = 128
H = 16
C = 32
K = 10

NC = 2
NS = 16
HALF = C // NC

NPAD = 10240
SLICE = NPAD // NS
TRASH = N

CHUNK = 128
CHUNKS = 160
EPT = CHUNK * CHUNKS
PADE = NS * EPT

_mesh = plsc.VectorSubcoreMesh(core_axis_name="c", subcore_axis_name="s")
_sc_params = pltpu.CompilerParams(use_tc_tiling_on_sc=False)


def _deg_body(src_hbm, dst_hbm, deg_hbm, idx_v, ones_v, zbuf, obuf, acc_sh,
              dsem):
    c = lax.axis_index("c")
    s = lax.axis_index("s")
    base = s * SLICE

    @pl.loop(0, SLICE)
    def _(i):
        zbuf[i] = jnp.zeros((HALF,), jnp.float32)

    @pl.loop(0, CHUNK)
    def _(i):
        ones_v[i] = jnp.ones((HALF,), jnp.float32)

    @pl.when(c == 0)
    def _():
        pltpu.sync_copy(src_hbm.at[s], idx_v)

    @pl.when(c == 1)
    def _():
        pltpu.sync_copy(dst_hbm.at[s], idx_v)

    pltpu.sync_copy(zbuf, acc_sh.at[pl.ds(base, SLICE)])
    plsc.subcore_barrier()

    @pl.loop(0, CHUNKS)
    def _(j):
        pltpu.async_copy(ones_v, acc_sh.at[idx_v.at[j]], dsem, add=True)

    @pl.loop(0, CHUNKS)
    def _(j):
        pltpu.make_async_copy(ones_v, acc_sh.at[idx_v.at[j]], dsem).wait()

    plsc.subcore_barrier()
    pltpu.sync_copy(acc_sh.at[pl.ds(base, SLICE)], obuf)
    pltpu.sync_copy(obuf, deg_hbm.at[c, pl.ds(base, SLICE)])


@jax.jit
def _deg_call(src_t, dst_t):
    return pl.kernel(
        _deg_body,
        out_type=jax.ShapeDtypeStruct((NC, NPAD, HALF), jnp.float32),
        mesh=_mesh,
        compiler_params=_sc_params,
        scratch_types=[
            pltpu.VMEM((CHUNKS, CHUNK), jnp.int32),
            pltpu.VMEM((CHUNK, HALF), jnp.float32),
            pltpu.VMEM((SLICE, HALF), jnp.float32),
            pltpu.VMEM((SLICE, HALF), jnp.float32),
            pltpu.VMEM_SHARED((NPAD, HALF), jnp.float32),
            pltpu.SemaphoreType.DMA,
        ],
    )(src_t, dst_t)


def _hl_body(x_ref, w0_ref, w1_ref, hl_ref):
    z1 = jax.nn.relu(jnp.dot(x_ref[...], w0_ref[...],
                             preferred_element_type=jnp.float32))
    hl_ref[...] = jnp.dot(z1, w1_ref[...], preferred_element_type=jnp.float32)


@jax.jit
def _hl_call(xp, W0, W1):
    nb = 2048
    grid = NPAD // nb
    return pl.pallas_call(
        _hl_body,
        grid=(grid,),
        in_specs=[
            pl.BlockSpec((nb, D), lambda i: (i, 0)),
            pl.BlockSpec((D, H), lambda i: (0, 0)),
            pl.BlockSpec((H, C), lambda i: (0, 0)),
        ],
        out_specs=pl.BlockSpec((nb, C), lambda i: (i, 0)),
        out_shape=jax.ShapeDtypeStruct((NPAD, C), jnp.float32),
    )(xp, W0, W1)


def _prep_body(deg_ref, hl_ref, s0_ref, dd_ref, db_ref):
    dsrc = lax.rsqrt(jnp.clip(deg_ref[0][:, 0:1], 1.0))
    ddst = lax.rsqrt(jnp.clip(deg_ref[1][:, 0:1], 1.0))
    s0_ref[...] = hl_ref[...] * dsrc
    ones = jnp.ones((1, HALF), jnp.float32)
    dd_ref[...] = ddst * ones
    db_ref[...] = (dsrc * ddst) * ones


@jax.jit
def _prep_call(degs, hl):
    nb = 2048
    grid = NPAD // nb
    return pl.pallas_call(
        _prep_body,
        grid=(grid,),
        in_specs=[
            pl.BlockSpec((NC, nb, HALF), lambda i: (0, i, 0)),
            pl.BlockSpec((nb, C), lambda i: (i, 0)),
        ],
        out_specs=[
            pl.BlockSpec((nb, C), lambda i: (i, 0)),
            pl.BlockSpec((nb, HALF), lambda i: (i, 0)),
            pl.BlockSpec((nb, HALF), lambda i: (i, 0)),
        ],
        out_shape=[
            jax.ShapeDtypeStruct((NPAD, C), jnp.float32),
            jax.ShapeDtypeStruct((NPAD, HALF), jnp.float32),
            jax.ShapeDtypeStruct((NPAD, HALF), jnp.float32),
        ],
    )(degs, hl)


GCHUNK = 256
GCHUNKS = EPT // GCHUNK
NBUF = 5
NBLK = GCHUNKS // NBUF
assert NBUF * NBLK == GCHUNKS


def _hops_body(srcg_hbm, dst_hbm, s0_hbm, dd_hbm, db_hbm, p_hbm,
               sidx, didx, rows, accb, pbuf, dd, db, zbuf, acc_sh, tab_sh,
               gsem, ssem):
    c = lax.axis_index("c")
    s = lax.axis_index("s")
    base = s * SLICE

    pltpu.sync_copy(srcg_hbm.at[s], sidx)
    pltpu.sync_copy(dst_hbm.at[s], didx)
    pltpu.sync_copy(dd_hbm.at[pl.ds(base, SLICE)], dd)
    pltpu.sync_copy(db_hbm.at[pl.ds(base, SLICE)], db)

    @pl.loop(0, SLICE // 2)
    def _(i):
        zbuf[i] = jnp.zeros((HALF,), jnp.float32)

    pltpu.sync_copy(s0_hbm.at[pl.ds(base, SLICE), pl.ds(c * HALF, HALF)], accb)
    pltpu.sync_copy(accb, tab_sh.at[pl.ds(base, SLICE)])

    def zero_acc():
        pltpu.sync_copy(zbuf, acc_sh.at[pl.ds(base, SLICE // 2)])
        pltpu.sync_copy(zbuf, acc_sh.at[pl.ds(base + SLICE // 2, SLICE // 2)])

    zero_acc()
    plsc.subcore_barrier()

    for k in range(1, K + 1):
        tab = tab_sh

        for b in range(NBUF):
            pltpu.async_copy(tab.at[sidx.at[b]], rows.at[b], gsem.at[b])

        @pl.loop(0, NBLK)
        def _(blk):
            j0 = blk * NBUF
            for b in range(NBUF):
                pltpu.make_async_copy(
                    tab.at[sidx.at[j0 + b]], rows.at[b], gsem.at[b]).wait()
                for h in range(2):
                    pltpu.async_copy(
                        rows.at[b].at[pl.ds(h * CHUNK, CHUNK)],
                        acc_sh.at[didx.at[2 * (j0 + b) + h]], ssem.at[b],
                        add=True)
            for b in range(NBUF):
                for h in range(2):
                    pltpu.make_async_copy(
                        rows.at[b].at[pl.ds(h * CHUNK, CHUNK)],
                        acc_sh.at[didx.at[2 * (j0 + b) + h]],
                        ssem.at[b]).wait()

                @pl.when(blk < NBLK - 1)
                def _():
                    pltpu.async_copy(
                        tab.at[sidx.at[j0 + NBUF + b]], rows.at[b],
                        gsem.at[b])

        plsc.subcore_barrier()

        pltpu.sync_copy(acc_sh.at[pl.ds(base, SLICE)], accb)

        @pl.loop(0, SLICE)
        def _(i):
            a = accb[i]
            pbuf[i] = a * dd[i]
            accb[i] = a * db[i]

        pltpu.sync_copy(
            pbuf, p_hbm.at[k - 1, pl.ds(base, SLICE), pl.ds(c * HALF, HALF)])
        pltpu.sync_copy(accb, tab_sh.at[pl.ds(base, SLICE)])
        zero_acc()
        plsc.subcore_barrier()


@jax.jit
def _hops_call(src_g, dst_t, s0, dd16, db16):
    return pl.kernel(
        _hops_body,
        out_type=jax.ShapeDtypeStruct((K, NPAD, C), jnp.float32),
        mesh=_mesh,
        compiler_params=_sc_params,
        scratch_types=[
            pltpu.VMEM((GCHUNKS, GCHUNK), jnp.int32),
            pltpu.VMEM((CHUNKS, CHUNK), jnp.int32),
            pltpu.VMEM((NBUF, GCHUNK, HALF), jnp.float32),
            pltpu.VMEM((SLICE, HALF), jnp.float32),
            pltpu.VMEM((SLICE, HALF), jnp.float32),
            pltpu.VMEM((SLICE, HALF), jnp.float32),
            pltpu.VMEM((SLICE, HALF), jnp.float32),
            pltpu.VMEM((SLICE // 2, HALF), jnp.float32),
            pltpu.VMEM_SHARED((NPAD, HALF), jnp.float32),
            pltpu.VMEM_SHARED((NPAD, HALF), jnp.float32),
            pltpu.SemaphoreType.DMA((NBUF,)),
            pltpu.SemaphoreType.DMA((NBUF,)),
        ],
    )(src_g, dst_t, s0, dd16, db16)


def _attn_body(hl_ref, p_ref, wa_ref, m_ref, mt_ref, out_ref):
    wa = wa_ref[...]
    gsum = m_ref[...]
    gbcast = mt_ref[...]
    zs = [hl_ref[...]] + [p_ref[k] for k in range(K)]
    a = [jnp.dot(z * wa, gsum, preferred_element_type=jnp.float32)
         for z in zs]
    m = a[0]
    for k in range(1, K + 1):
        m = jnp.maximum(m, a[k])
    e = [jnp.exp(ak - m) for ak in a]
    s = e[0]
    for k in range(1, K + 1):
        s = s + e[k]
    inv = 1.0 / s
    out = jnp.zeros_like(zs[0])
    for k in range(K + 1):
        coefx = jnp.dot(e[k] * inv, gbcast,
                        preferred_element_type=jnp.float32)
        out = out + zs[k] * coefx
    out_ref[...] = out


@jax.jit
def _attn_call(hl, P, wap):
    n4 = NPAD // 4
    nb = n4 // 2
    hl4 = hl.reshape(n4, 4 * C)
    p4 = P.reshape(K, n4, 4 * C)
    wa4 = wap.reshape(n4, 4 * C)
    g = jnp.arange(4 * C, dtype=jnp.int32) // C
    gsum = (g[:, None] == jnp.arange(4, dtype=jnp.int32)[None, :]
            ).astype(jnp.float32)
    out4 = pl.pallas_call(
        _attn_body,
        grid=(2,),
        in_specs=[
            pl.BlockSpec((nb, 4 * C), lambda i: (i, 0)),
            pl.BlockSpec((K, nb, 4 * C), lambda i: (0, i, 0)),
            pl.BlockSpec((nb, 4 * C), lambda i: (i, 0)),
            pl.BlockSpec((4 * C, 4), lambda i: (0, 0)),
            pl.BlockSpec((4, 4 * C), lambda i: (0, 0)),
        ],
        out_specs=pl.BlockSpec((nb, 4 * C), lambda i: (i, 0)),
        out_shape=jax.ShapeDtypeStruct((n4, 4 * C), jnp.float32),
    )(hl4, p4, wa4, gsum, gsum.T)
    return out4.reshape(NPAD, C)


def kernel(x, edge_index, W0, W1, W_attn):
    src = edge_index[0].astype(jnp.int32)
    dst = edge_index[1].astype(jnp.int32)
    pad = PADE - E
    srcp = jnp.concatenate([src, jnp.full((pad,), TRASH, jnp.int32)])
    dstp = jnp.concatenate([dst, jnp.full((pad,), TRASH, jnp.int32)])
    src_t = srcp.reshape(NS, CHUNKS, CHUNK)
    src_g = srcp.reshape(NS, GCHUNKS, GCHUNK)
    dst_t = dstp.reshape(NS, CHUNKS, CHUNK)

    xp = jnp.pad(x, ((0, NPAD - N), (0, 0)))
    wap = jnp.pad(W_attn, ((0, NPAD - N), (0, 0)))

    degs = _deg_call(src_t, dst_t)
    hl = _hl_call(xp, W0, W1)
    s0, dd16, db16 = _prep_call(degs, hl)
    P = _hops_call(src_g, dst_t, s0, dd16, db16)
    logits = _attn_call(hl, P, wap)
    return logits[:N]
